# baseline (device time: 47579 ns/iter reference)
import jax
import jax.numpy as jnp
from jax import lax
from jax.experimental import pallas as pl
from jax.experimental.pallas import tpu as pltpu

N_DEV = 4
N_TOK = 512
D_IN = 256
D_OUT = 512
E_LOCAL = 2
CAP = 51


def kernel(x, router_W, route_idx, expert_W):
    def body(x_ref, rw_ref, idx_ref, ew_ref, out_ref, comm_ref, send_sems, recv_sems):
        my = lax.axis_index("i")
        left = (my - 1) % N_DEV
        right = (my + 1) % N_DEV

        barrier_sem = pltpu.get_barrier_semaphore()
        for nbr in (left, right):
            pl.semaphore_signal(
                barrier_sem, inc=1,
                device_id=(nbr,), device_id_type=pl.DeviceIdType.MESH,
            )
        pl.semaphore_wait(barrier_sem, 2)

        row = lax.broadcasted_iota(jnp.int32, (N_TOK, N_TOK), 0)
        col = lax.broadcasted_iota(jnp.int32, (N_TOK, N_TOK), 1)
        tril = (col < row).astype(jnp.float32)
        e_iota = lax.broadcasted_iota(jnp.int32, (N_TOK, E_LOCAL), 1)
        onehot = (idx_ref[:, :] == E_LOCAL * my + e_iota).astype(jnp.float32)
        prior = jnp.dot(tril, onehot, preferred_element_type=jnp.float32)
        mask = jnp.where((prior < CAP) & (onehot > 0.5), 1.0, 0.0)

        p = jnp.dot(x_ref[:, :] * mask[:, 0:1], ew_ref[0],
                    preferred_element_type=jnp.float32)
        p = p + jnp.dot(x_ref[:, :] * mask[:, 1:2], ew_ref[1],
                        preferred_element_type=jnp.float32)

        out_ref[:, :] = p
        comm_ref[0, :, :] = p

        for h in range(N_DEV - 1):
            rdma = pltpu.make_async_remote_copy(
                src_ref=comm_ref.at[h],
                dst_ref=comm_ref.at[h + 1],
                send_sem=send_sems.at[h],
                recv_sem=recv_sems.at[h],
                device_id=(right,),
                device_id_type=pl.DeviceIdType.MESH,
            )
            rdma.start()
            rdma.wait()
            out_ref[:, :] = out_ref[:, :] + comm_ref[h + 1, :, :]

    return pl.pallas_call(
        body,
        out_shape=jax.ShapeDtypeStruct((N_TOK, D_OUT), jnp.float32),
        in_specs=[pl.BlockSpec(memory_space=pltpu.VMEM)] * 4,
        out_specs=pl.BlockSpec(memory_space=pltpu.VMEM),
        scratch_shapes=[
            pltpu.VMEM((N_DEV, N_TOK, D_OUT), jnp.float32),
            pltpu.SemaphoreType.DMA((N_DEV - 1,)),
            pltpu.SemaphoreType.DMA((N_DEV - 1,)),
        ],
        compiler_params=pltpu.CompilerParams(collective_id=0),
    )(x, router_W, route_idx, expert_W)


# device time: 17620 ns/iter; 2.7003x vs baseline; 2.7003x over previous
import jax
import jax.numpy as jnp
from jax import lax
from jax.experimental import pallas as pl
from jax.experimental.pallas import tpu as pltpu

N_DEV = 4
N_TOK = 512
D_IN = 256
D_OUT = 512
E_TOT = 8
E_LOCAL = 2
CAP = 51
SLOT_E = 64
SLOT_LOC = E_LOCAL * SLOT_E
SLOT_TOT = N_DEV * SLOT_LOC


def kernel(x, router_W, route_idx, expert_W):
    def body(x_ref, rw_ref, idx_ref, ew_ref, out_ref, comm_ref, ss, rs):
        my = lax.axis_index("i")
        left = (my - 1) % N_DEV
        right = (my + 1) % N_DEV

        barrier_sem = pltpu.get_barrier_semaphore()
        for nbr in (left, right):
            pl.semaphore_signal(
                barrier_sem, inc=1,
                device_id=(nbr,), device_id_type=pl.DeviceIdType.MESH,
            )
        pl.semaphore_wait(barrier_sem, 2)

        row = lax.broadcasted_iota(jnp.int32, (N_TOK, N_TOK), 0)
        col = lax.broadcasted_iota(jnp.int32, (N_TOK, N_TOK), 1)
        tril = (col < row).astype(jnp.float32)
        e_iota = lax.broadcasted_iota(jnp.int32, (N_TOK, E_TOT), 1)
        route = idx_ref[:, :]
        onehot = (route == e_iota).astype(jnp.float32)
        prior = jnp.dot(tril, onehot, preferred_element_type=jnp.float32)
        prior_sel = jnp.sum(prior * onehot, axis=1, keepdims=True)
        accept = prior_sel < CAP
        prior_i = prior_sel.astype(jnp.int32)
        slot_global = SLOT_E * route + prior_i

        s_iota = lax.broadcasted_iota(jnp.int32, (N_TOK, SLOT_LOC), 1)
        p_loc = jnp.where(
            (slot_global - SLOT_LOC * my == s_iota) & accept, 1.0, 0.0
        )

        xc = lax.dot_general(
            p_loc, x_ref[:, :],
            dimension_numbers=(((0,), (0,)), ((), ())),
            preferred_element_type=jnp.float32,
        )

        comm_ref[0, 0:SLOT_E, :] = jnp.dot(
            xc[0:SLOT_E, :], ew_ref[0], preferred_element_type=jnp.float32)
        comm_ref[0, SLOT_E:SLOT_LOC, :] = jnp.dot(
            xc[SLOT_E:SLOT_LOC, :], ew_ref[1], preferred_element_type=jnp.float32)

        r1 = pltpu.make_async_remote_copy(
            src_ref=comm_ref.at[0], dst_ref=comm_ref.at[3],
            send_sem=ss.at[0], recv_sem=rs.at[0],
            device_id=(right,), device_id_type=pl.DeviceIdType.MESH,
        )
        l1 = pltpu.make_async_remote_copy(
            src_ref=comm_ref.at[0], dst_ref=comm_ref.at[1],
            send_sem=ss.at[1], recv_sem=rs.at[1],
            device_id=(left,), device_id_type=pl.DeviceIdType.MESH,
        )
        r1.start()
        l1.start()

        j_iota = lax.broadcasted_iota(jnp.int32, (N_TOK, SLOT_TOT), 1)
        slot_rel = (slot_global - SLOT_LOC * my) % SLOT_TOT
        p_rel = jnp.where((slot_rel == j_iota) & accept, 1.0, 0.0)

        r1.wait_recv()
        r2 = pltpu.make_async_remote_copy(
            src_ref=comm_ref.at[3], dst_ref=comm_ref.at[2],
            send_sem=ss.at[2], recv_sem=rs.at[2],
            device_id=(right,), device_id_type=pl.DeviceIdType.MESH,
        )
        r2.start()
        l1.wait_recv()
        r2.wait_recv()

        g = comm_ref[:, :, :].reshape(SLOT_TOT, D_OUT)
        out_ref[:, :] = jnp.dot(p_rel, g, preferred_element_type=jnp.float32)

        r1.wait_send()
        l1.wait_send()
        r2.wait_send()

    return pl.pallas_call(
        body,
        out_shape=jax.ShapeDtypeStruct((N_TOK, D_OUT), jnp.float32),
        in_specs=[pl.BlockSpec(memory_space=pltpu.VMEM)] * 4,
        out_specs=pl.BlockSpec(memory_space=pltpu.VMEM),
        scratch_shapes=[
            pltpu.VMEM((N_DEV, SLOT_LOC, D_OUT), jnp.float32),
            pltpu.SemaphoreType.DMA((3,)),
            pltpu.SemaphoreType.DMA((3,)),
        ],
        compiler_params=pltpu.CompilerParams(collective_id=0),
    )(x, router_W, route_idx, expert_W)


# device time: 12696 ns/iter; 3.7476x vs baseline; 1.3878x over previous
import jax
import jax.numpy as jnp
from jax import lax
from jax.experimental import pallas as pl
from jax.experimental.pallas import tpu as pltpu

N_DEV = 4
N_TOK = 512
D_IN = 256
D_OUT = 512
E_TOT = 8
E_LOCAL = 2
CAP = 51
SLOT_E = 56
SLOT_LOC = E_LOCAL * SLOT_E
SLOT_TOT = N_DEV * SLOT_LOC
BF = jnp.bfloat16


def kernel(x, router_W, route_idx, expert_W):
    def body(x_ref, rw_ref, idx_ref, ew_ref, out_ref, comm_ref, ss, rs):
        my = lax.axis_index("i")
        left = (my - 1) % N_DEV
        right = (my + 1) % N_DEV
        diag = (my + 2) % N_DEV

        barrier_sem = pltpu.get_barrier_semaphore()
        for nbr in (left, right, diag):
            pl.semaphore_signal(
                barrier_sem, inc=1,
                device_id=(nbr,), device_id_type=pl.DeviceIdType.MESH,
            )

        e_iota = lax.broadcasted_iota(jnp.int32, (N_TOK, E_TOT), 1)
        route = idx_ref[:, :]
        onehot = (route == e_iota).astype(jnp.float32)
        cum = onehot
        for sh in (1, 2, 4, 8, 16, 32, 64, 128, 256):
            cum = cum + jnp.concatenate(
                [jnp.zeros((sh, E_TOT), jnp.float32), cum[:-sh, :]], axis=0)
        prior_sel = jnp.sum(onehot * cum, axis=1, keepdims=True) - 1.0
        accept = prior_sel < CAP
        prior_i = prior_sel.astype(jnp.int32)
        slot_global = SLOT_E * route + prior_i

        s_iota = lax.broadcasted_iota(jnp.int32, (N_TOK, SLOT_LOC), 1)
        p_loc = ((slot_global - SLOT_LOC * my == s_iota) & accept).astype(BF)

        xc = lax.dot_general(
            p_loc, x_ref[:, :].astype(BF),
            dimension_numbers=(((0,), (0,)), ((), ())),
            preferred_element_type=jnp.float32,
        ).astype(BF)

        comm_ref[0, 0:SLOT_E, :] = jnp.dot(
            xc[0:SLOT_E, :], ew_ref[0].astype(BF),
            preferred_element_type=jnp.float32).astype(BF)
        comm_ref[0, SLOT_E:SLOT_LOC, :] = jnp.dot(
            xc[SLOT_E:SLOT_LOC, :], ew_ref[1].astype(BF),
            preferred_element_type=jnp.float32).astype(BF)

        pl.semaphore_wait(barrier_sem, 3)

        r1 = pltpu.make_async_remote_copy(
            src_ref=comm_ref.at[0], dst_ref=comm_ref.at[3],
            send_sem=ss.at[0], recv_sem=rs.at[0],
            device_id=(right,), device_id_type=pl.DeviceIdType.MESH,
        )
        l1 = pltpu.make_async_remote_copy(
            src_ref=comm_ref.at[0], dst_ref=comm_ref.at[1],
            send_sem=ss.at[1], recv_sem=rs.at[1],
            device_id=(left,), device_id_type=pl.DeviceIdType.MESH,
        )
        d1 = pltpu.make_async_remote_copy(
            src_ref=comm_ref.at[0], dst_ref=comm_ref.at[2],
            send_sem=ss.at[2], recv_sem=rs.at[2],
            device_id=(diag,), device_id_type=pl.DeviceIdType.MESH,
        )
        r1.start()
        l1.start()
        d1.start()

        j_iota = lax.broadcasted_iota(jnp.int32, (N_TOK, SLOT_TOT), 1)
        slot_rel = (slot_global - SLOT_LOC * my) % SLOT_TOT
        p_rel = ((slot_rel == j_iota) & accept).astype(BF)

        def scatter(k):
            return lax.dot_general(
                p_rel[:, k * SLOT_LOC:(k + 1) * SLOT_LOC], comm_ref[k, :, :],
                dimension_numbers=(((1,), (0,)), ((), ())),
                preferred_element_type=jnp.float32,
            )

        acc = scatter(0)
        r1.wait_recv()
        acc = acc + scatter(3)
        l1.wait_recv()
        acc = acc + scatter(1)
        d1.wait_recv()
        acc = acc + scatter(2)
        out_ref[:, :] = acc

        r1.wait_send()
        l1.wait_send()
        d1.wait_send()

    return pl.pallas_call(
        body,
        out_shape=jax.ShapeDtypeStruct((N_TOK, D_OUT), jnp.float32),
        in_specs=[pl.BlockSpec(memory_space=pltpu.VMEM)] * 4,
        out_specs=pl.BlockSpec(memory_space=pltpu.VMEM),
        scratch_shapes=[
            pltpu.VMEM((N_DEV, SLOT_LOC, D_OUT), BF),
            pltpu.SemaphoreType.DMA((3,)),
            pltpu.SemaphoreType.DMA((3,)),
        ],
        compiler_params=pltpu.CompilerParams(collective_id=0),
    )(x, router_W, route_idx, expert_W)
